# baseline (speedup 1.0000x reference)
import functools

import jax
import jax.numpy as jnp
from jax import lax
from jax.experimental import pallas as pl
from jax.experimental.pallas import tpu as pltpu
from jax.experimental.pallas import tpu_sc as plsc

D_MODEL = 128
NUM_CORES = 2
NUM_SUBCORES = 16
NW = NUM_CORES * NUM_SUBCORES
CHUNK = 128
N_BUF = 5


@jax.jit
def _gather_flat(x_r, pe):
    n_chunks = x_r.shape[1]
    d = pe.shape[1]
    assert n_chunks % N_BUF == 0
    mesh = plsc.VectorSubcoreMesh(
        core_axis_name="c",
        subcore_axis_name="s",
        num_cores=NUM_CORES,
        num_subcores=NUM_SUBCORES,
    )

    @functools.partial(
        pl.kernel,
        mesh=mesh,
        out_type=jax.ShapeDtypeStruct((NW, n_chunks, CHUNK, d), jnp.float32),
        scratch_types=[
            pltpu.VMEM((n_chunks, CHUNK), jnp.int32),
            [pltpu.VMEM((CHUNK, d), jnp.float32) for _ in range(N_BUF)],
            [pltpu.SemaphoreType.DMA for _ in range(N_BUF)],
            [pltpu.SemaphoreType.DMA for _ in range(N_BUF)],
        ],
    )
    def k(x_hbm, pe_hbm, out_hbm, idx_v, bufs, gsems, wsems):
        wid = lax.axis_index("s") * NUM_CORES + lax.axis_index("c")
        pltpu.sync_copy(x_hbm.at[wid], idx_v)
        for b in range(N_BUF):
            pltpu.async_copy(pe_hbm.at[idx_v.at[b]], bufs[b], gsems[b])

        @pl.loop(0, n_chunks, step=N_BUF)
        def body(j):
            for b in range(N_BUF):
                pltpu.make_async_copy(
                    pe_hbm.at[idx_v.at[j + b]], bufs[b], gsems[b]
                ).wait()
                pltpu.async_copy(bufs[b], out_hbm.at[wid, j + b], wsems[b])
            for b in range(N_BUF):
                @pl.when(j + N_BUF + b < n_chunks)
                def _():
                    pltpu.make_async_copy(
                        bufs[b], out_hbm.at[wid, j + b], wsems[b]
                    ).wait()
                    pltpu.async_copy(
                        pe_hbm.at[idx_v.at[j + N_BUF + b]], bufs[b], gsems[b]
                    )

        for b in range(N_BUF):
            last = n_chunks - N_BUF + b
            pltpu.make_async_copy(bufs[b], out_hbm.at[wid, last], wsems[b]).wait()

    return k(x_r, pe)


def kernel(x, pe):
    b, s = x.shape
    total = b * s
    assert total % (NW * CHUNK) == 0
    n_chunks = total // (NW * CHUNK)
    x_r = x.reshape(NW, n_chunks, CHUNK)
    out = _gather_flat(x_r, pe)
    return out.reshape(b, s, pe.shape[1])


# speedup vs baseline: 1.1274x; 1.1274x over previous
import functools

import jax
import jax.numpy as jnp
from jax import lax
from jax.experimental import pallas as pl
from jax.experimental.pallas import tpu as pltpu
from jax.experimental.pallas import tpu_sc as plsc

D_MODEL = 128
NUM_CORES = 2
NUM_SUBCORES = 16
NW = NUM_CORES * NUM_SUBCORES
CHUNK = 128
N_BUF = 2


@jax.jit
def _gather_flat(x_r, pe):
    n_chunks = x_r.shape[1]
    d = pe.shape[1]
    assert n_chunks % N_BUF == 0
    mesh = plsc.VectorSubcoreMesh(
        core_axis_name="c",
        subcore_axis_name="s",
        num_cores=NUM_CORES,
        num_subcores=NUM_SUBCORES,
    )

    v = pe.shape[0]

    @functools.partial(
        pl.kernel,
        mesh=mesh,
        out_type=jax.ShapeDtypeStruct((NW, n_chunks, CHUNK, d), jnp.float32),
        scratch_types=[
            pltpu.VMEM((n_chunks, CHUNK), jnp.int32),
            pltpu.VMEM_SHARED((v, d), jnp.float32),
            [pltpu.VMEM((CHUNK, d), jnp.float32) for _ in range(N_BUF)],
            [pltpu.SemaphoreType.DMA for _ in range(N_BUF)],
            [pltpu.SemaphoreType.DMA for _ in range(N_BUF)],
        ],
    )
    def k(x_hbm, pe_hbm, out_hbm, idx_v, pe_sp, bufs, gsems, wsems):
        wid = lax.axis_index("s") * NUM_CORES + lax.axis_index("c")
        sid = lax.axis_index("s")
        rows_per_sub = v // NUM_SUBCORES
        pltpu.sync_copy(
            pe_hbm.at[pl.ds(sid * rows_per_sub, rows_per_sub)],
            pe_sp.at[pl.ds(sid * rows_per_sub, rows_per_sub)],
        )
        pltpu.sync_copy(x_hbm.at[wid], idx_v)
        plsc.subcore_barrier()

        for b in range(N_BUF):
            pltpu.async_copy(pe_sp.at[idx_v.at[b]], bufs[b], gsems[b])

        @pl.loop(0, n_chunks, step=N_BUF)
        def body(j):
            for b in range(N_BUF):
                pltpu.make_async_copy(
                    pe_sp.at[idx_v.at[j + b]], bufs[b], gsems[b]
                ).wait()
                pltpu.async_copy(bufs[b], out_hbm.at[wid, j + b], wsems[b])
            for b in range(N_BUF):
                @pl.when(j + N_BUF + b < n_chunks)
                def _():
                    pltpu.make_async_copy(
                        bufs[b], out_hbm.at[wid, j + b], wsems[b]
                    ).wait()
                    pltpu.async_copy(
                        pe_sp.at[idx_v.at[j + N_BUF + b]], bufs[b], gsems[b]
                    )

        for b in range(N_BUF):
            last = n_chunks - N_BUF + b
            pltpu.make_async_copy(bufs[b], out_hbm.at[wid, last], wsems[b]).wait()

    return k(x_r, pe)


def kernel(x, pe):
    b, s = x.shape
    total = b * s
    assert total % (NW * CHUNK) == 0
    n_chunks = total // (NW * CHUNK)
    x_r = x.reshape(NW, n_chunks, CHUNK)
    out = _gather_flat(x_r, pe)
    return out.reshape(b, s, pe.shape[1])
